# Initial kernel scaffold; baseline (speedup 1.0000x reference)
import jax
import jax.numpy as jnp
from jax.experimental import pallas as pl


def kernel(vel_norm, x, edges, vel, edge_attr, params):
    raise NotImplementedError("write your pallas kernel here")


# speedup vs baseline: 9.1412x; 9.1412x over previous
import functools

import jax
import jax.numpy as jnp
from jax import lax
from jax.experimental import pallas as pl
from jax.experimental.pallas import tpu as pltpu
from jax.experimental.pallas import tpu_sc as plsc

NC = 2
NS = 16
NW = NC * NS
LANES = 16

DIM = 3
N_NODES = 100000
N_PAD = 102400
N_EDGES = 1600000
NF = 32
EA = 4

G_CHUNKS = 10
G_PER = N_EDGES // G_CHUNKS
G_SUB = 4000
S_PER = N_EDGES // NW
S_SUB = 10000
ACC_FLAT = DIM * N_PAD
ACC_CHUNK = ACC_FLAT // NS
CACC_FLAT = N_PAD
CACC_CHUNK = CACC_FLAT // NS

EB = 6400
NB = 6400

_mesh = plsc.VectorSubcoreMesh(core_axis_name="c", subcore_axis_name="s")


def _wid():
    return lax.axis_index("s") * NC + lax.axis_index("c")


@functools.partial(
    pl.kernel,
    out_type=jax.ShapeDtypeStruct((DIM * N_EDGES,), jnp.float32),
    mesh=_mesh,
    compiler_params=pltpu.CompilerParams(needs_layout_passes=False),
    scratch_types=[
        pltpu.VMEM((N_PAD,), jnp.float32),
        pltpu.VMEM((G_SUB,), jnp.int32),
        pltpu.VMEM((G_SUB,), jnp.int32),
        pltpu.VMEM((G_SUB,), jnp.float32),
    ],
)
def _sc_gather_diff(xTf, rowi, coli, out, table, ri, ci, dif):
    wid = _wid()
    comp = wid // G_CHUNKS
    j = wid % G_CHUNKS

    @pl.when(comp < DIM)
    def _():
        tbase = pl.multiple_of(comp * N_PAD, 8)
        pltpu.sync_copy(xTf.at[pl.ds(tbase, N_PAD)], table)

        def sub_body(s, carry):
            base = j * G_PER + s * G_SUB
            pltpu.sync_copy(rowi.at[pl.ds(base, G_SUB)], ri)
            pltpu.sync_copy(coli.at[pl.ds(base, G_SUB)], ci)

            def gath(i, c2):
                off = i * LANES
                idr = ri[pl.ds(off, LANES)]
                idc = ci[pl.ds(off, LANES)]
                vr = plsc.load_gather(table, [idr])
                vc = plsc.load_gather(table, [idc])
                dif[pl.ds(off, LANES)] = vr - vc
                return c2

            lax.fori_loop(0, G_SUB // LANES, gath, 0)
            obase = pl.multiple_of(comp * N_EDGES + base, 8)
            pltpu.sync_copy(dif, out.at[pl.ds(obase, G_SUB)])
            return carry

        lax.fori_loop(0, G_PER // G_SUB, sub_body, 0)


@functools.partial(
    pl.kernel,
    out_type=jax.ShapeDtypeStruct((NC * DIM * N_PAD,), jnp.float32),
    mesh=_mesh,
    compiler_params=pltpu.CompilerParams(needs_layout_passes=False),
    scratch_types=[
        pltpu.VMEM_SHARED((ACC_FLAT,), jnp.float32),
        pltpu.VMEM((ACC_CHUNK,), jnp.float32),
        pltpu.VMEM((S_SUB,), jnp.int32),
        pltpu.VMEM((S_SUB,), jnp.int32),
        pltpu.VMEM((S_SUB,), jnp.float32),
    ],
)
def _sc_scatter(emTf, rowi, out, acc, zbuf, ri, off, ev):
    cc = lax.axis_index("c")
    sid = lax.axis_index("s")

    def zfill(i, c2):
        zbuf[pl.ds(i * LANES, LANES)] = jnp.zeros((LANES,), jnp.float32)
        return c2

    lax.fori_loop(0, ACC_CHUNK // LANES, zfill, 0)
    pltpu.sync_copy(zbuf, acc.at[pl.ds(sid * ACC_CHUNK, ACC_CHUNK)])
    plsc.subcore_barrier()

    def sub_body(s, carry):
        base = (cc * NS + sid) * S_PER + s * S_SUB
        pltpu.sync_copy(rowi.at[pl.ds(base, S_SUB)], ri)
        for c in range(DIM):
            def obody(i, c2):
                o = i * LANES
                off[pl.ds(o, LANES)] = ri[pl.ds(o, LANES)] + c * N_PAD
                return c2

            lax.fori_loop(0, S_SUB // LANES, obody, 0)
            ebase = pl.multiple_of(c * N_EDGES + base, 8)
            pltpu.sync_copy(emTf.at[pl.ds(ebase, S_SUB)], ev)
            pltpu.sync_copy(ev, acc.at[off], add=True)
        return carry

    lax.fori_loop(0, S_PER // S_SUB, sub_body, 0)
    plsc.subcore_barrier()

    abase = pl.multiple_of(sid * ACC_CHUNK, 8)
    obase = pl.multiple_of(cc * ACC_FLAT + sid * ACC_CHUNK, 8)
    pltpu.sync_copy(acc.at[pl.ds(abase, ACC_CHUNK)], zbuf)
    pltpu.sync_copy(zbuf, out.at[pl.ds(obase, ACC_CHUNK)])


@functools.partial(
    pl.kernel,
    out_type=jax.ShapeDtypeStruct((NC * N_PAD,), jnp.float32),
    mesh=_mesh,
    compiler_params=pltpu.CompilerParams(needs_layout_passes=False),
    scratch_types=[
        pltpu.VMEM_SHARED((CACC_FLAT,), jnp.float32),
        pltpu.VMEM((CACC_CHUNK,), jnp.float32),
        pltpu.VMEM((S_SUB,), jnp.int32),
        pltpu.VMEM((S_SUB,), jnp.float32),
    ],
)
def _sc_count(rowi, out, acc, zbuf, ri, ones):
    cc = lax.axis_index("c")
    sid = lax.axis_index("s")

    def zfill(i, c2):
        zbuf[pl.ds(i * LANES, LANES)] = jnp.zeros((LANES,), jnp.float32)
        return c2

    lax.fori_loop(0, CACC_CHUNK // LANES, zfill, 0)
    pltpu.sync_copy(zbuf, acc.at[pl.ds(sid * CACC_CHUNK, CACC_CHUNK)])

    def ones_body(i, c2):
        ones[pl.ds(i * LANES, LANES)] = jnp.ones((LANES,), jnp.float32)
        return c2

    lax.fori_loop(0, S_SUB // LANES, ones_body, 0)
    plsc.subcore_barrier()

    def sub_body(s, carry):
        base = (cc * NS + sid) * S_PER + s * S_SUB
        pltpu.sync_copy(rowi.at[pl.ds(base, S_SUB)], ri)
        pltpu.sync_copy(ones, acc.at[ri], add=True)
        return carry

    lax.fori_loop(0, S_PER // S_SUB, sub_body, 0)
    plsc.subcore_barrier()

    abase = pl.multiple_of(sid * CACC_CHUNK, 8)
    cbase = pl.multiple_of(cc * N_PAD + sid * CACC_CHUNK, 8)
    pltpu.sync_copy(acc.at[pl.ds(abase, CACC_CHUNK)], zbuf)
    pltpu.sync_copy(zbuf, out.at[pl.ds(cbase, CACC_CHUNK)])


def _edge_mlp_body(xd_ref, ea_ref, w1_ref, b1_ref, w2_ref, out_ref):
    xd = xd_ref[...]
    r2 = jnp.sum(xd * xd, axis=0, keepdims=True)
    radial = jnp.sqrt(r2)
    w1 = w1_ref[...]
    ea = ea_ref[...]
    h = w1[0][:, None] * radial
    for k in range(EA):
        h = h + w1[k + 1][:, None] * ea[k][None, :]
    h = h + b1_ref[...]
    h = h * jax.nn.sigmoid(h)
    eo = jnp.sum(h * w2_ref[...], axis=0, keepdims=True)
    eo = jnp.tanh(eo)
    out_ref[...] = xd * eo


def _edge_mlp(xd, eaT, w1, b1, w2):
    grid = N_EDGES // EB
    return pl.pallas_call(
        _edge_mlp_body,
        grid=(grid,),
        in_specs=[
            pl.BlockSpec((DIM, EB), lambda i: (0, i)),
            pl.BlockSpec((EA, EB), lambda i: (0, i)),
            pl.BlockSpec((1 + EA, NF), lambda i: (0, 0)),
            pl.BlockSpec((NF, 1), lambda i: (0, 0)),
            pl.BlockSpec((NF, 1), lambda i: (0, 0)),
        ],
        out_specs=pl.BlockSpec((DIM, EB), lambda i: (0, i)),
        out_shape=jax.ShapeDtypeStruct((DIM, N_EDGES), jnp.float32),
    )(xd, eaT, w1, b1, w2)


def _inv_cnt_body(part_ref, out_ref):
    p = part_ref[...]
    s = jnp.sum(p, axis=0, keepdims=True)
    out_ref[...] = 1.0 / jnp.maximum(s, 1.0)


def _inv_cnt(part):
    grid = N_PAD // NB
    return pl.pallas_call(
        _inv_cnt_body,
        grid=(grid,),
        in_specs=[pl.BlockSpec((NC, NB), lambda i: (0, i))],
        out_specs=pl.BlockSpec((1, NB), lambda i: (0, i)),
        out_shape=jax.ShapeDtypeStruct((1, N_PAD), jnp.float32),
    )(part)


def _update_body(xT_ref, part_ref, inv_ref, velT_ref, vn_ref,
                 vw1_ref, vb1_ref, vw2_ref, vb2_ref, out_ref):
    part = part_ref[...]
    agg = (part[0] + part[1]) * inv_ref[...]

    vn = vn_ref[...]
    h = vw1_ref[...][0][:, None] * vn + vb1_ref[...]
    h = h * jax.nn.sigmoid(h)
    vs = jnp.sum(h * vw2_ref[...], axis=0, keepdims=True) + vb2_ref[...]

    out_ref[...] = xT_ref[...] + agg + velT_ref[...] * vs


def _update(xT, part, inv, velT, vnT, vw1, vb1, vw2, vb2):
    grid = N_PAD // NB
    return pl.pallas_call(
        _update_body,
        grid=(grid,),
        in_specs=[
            pl.BlockSpec((DIM, NB), lambda i: (0, i)),
            pl.BlockSpec((NC, DIM, NB), lambda i: (0, 0, i)),
            pl.BlockSpec((1, NB), lambda i: (0, i)),
            pl.BlockSpec((DIM, NB), lambda i: (0, i)),
            pl.BlockSpec((1, NB), lambda i: (0, i)),
            pl.BlockSpec((1, NF), lambda i: (0, 0)),
            pl.BlockSpec((NF, 1), lambda i: (0, 0)),
            pl.BlockSpec((NF, 1), lambda i: (0, 0)),
            pl.BlockSpec((1, 1), lambda i: (0, 0)),
        ],
        out_specs=pl.BlockSpec((DIM, NB), lambda i: (0, i)),
        out_shape=jax.ShapeDtypeStruct((DIM, N_PAD), jnp.float32),
    )(xT, part, inv, velT, vnT, vw1, vb1, vw2, vb2)


def kernel(vel_norm, x, edges, vel, edge_attr, params):
    row = edges[0]
    col = edges[1]

    pad = N_PAD - N_NODES
    xT = jnp.pad(x.T, ((0, 0), (0, pad)))
    velT = jnp.pad(vel.T, ((0, 0), (0, pad)))
    vnT = jnp.pad(vel_norm.T, ((0, 0), (0, pad)))
    eaT = edge_attr.T

    cnt_part = _sc_count(row).reshape(NC, N_PAD)
    inv = _inv_cnt(cnt_part)

    for p in params:
        xd = _sc_gather_diff(xT.reshape(-1), row, col).reshape(DIM, N_EDGES)
        em = _edge_mlp(xd, eaT, p['pW1'], p['pb1'].reshape(NF, 1), p['pW2'])
        part = _sc_scatter(em.reshape(-1), row).reshape(NC, DIM, N_PAD)
        xT = _update(xT, part, inv, velT, vnT,
                     p['vW1'], p['vb1'].reshape(NF, 1), p['vW2'],
                     p['vb2'].reshape(1, 1))

    return xT[:, :N_NODES].T


# speedup vs baseline: 10.6228x; 1.1621x over previous
import functools

import jax
import jax.numpy as jnp
from jax import lax
from jax.experimental import pallas as pl
from jax.experimental.pallas import tpu as pltpu
from jax.experimental.pallas import tpu_sc as plsc

NC = 2
NS = 16
NW = NC * NS
LANES = 16

DIM = 3
N_NODES = 100000
N_PAD = 102400
N_EDGES = 1600000
NF = 32
EA = 4

G_CHUNKS = 10
G_PER = N_EDGES // G_CHUNKS
G_SUB = 4000
S_PER = N_EDGES // NW
S_SUB = 10000
ACC_FLAT = DIM * N_PAD
ACC_CHUNK = ACC_FLAT // NS
CACC_FLAT = N_PAD
CACC_CHUNK = CACC_FLAT // NS

EB = 6400
NB = 6400

_mesh = plsc.VectorSubcoreMesh(core_axis_name="c", subcore_axis_name="s")


def _wid():
    return lax.axis_index("s") * NC + lax.axis_index("c")


@functools.partial(
    pl.kernel,
    out_type=jax.ShapeDtypeStruct((DIM * N_EDGES,), jnp.float32),
    mesh=_mesh,
    compiler_params=pltpu.CompilerParams(needs_layout_passes=False),
    scratch_types=[
        pltpu.VMEM((N_PAD,), jnp.float32),
        pltpu.VMEM((G_SUB,), jnp.int32),
        pltpu.VMEM((G_SUB,), jnp.int32),
        pltpu.VMEM((G_SUB,), jnp.int32),
        pltpu.VMEM((G_SUB,), jnp.int32),
        pltpu.VMEM((G_SUB,), jnp.float32),
        pltpu.VMEM((G_SUB,), jnp.float32),
        pltpu.SemaphoreType.DMA,
        pltpu.SemaphoreType.DMA,
        pltpu.SemaphoreType.DMA,
        pltpu.SemaphoreType.DMA,
    ],
)
def _sc_gather_diff(xTf, rowi, coli, out, table, ri0, ri1, ci0, ci1,
                    dif0, dif1, sin0, sin1, sout0, sout1):
    wid = _wid()
    comp = wid // G_CHUNKS
    j = wid % G_CHUNKS
    n_sub = G_PER // G_SUB
    ris = (ri0, ri1)
    cis = (ci0, ci1)
    difs = (dif0, dif1)
    sins = (sin0, sin1)
    souts = (sout0, sout1)

    @pl.when(comp < DIM)
    def _():
        tbase = pl.multiple_of(comp * N_PAD, 8)
        pltpu.sync_copy(xTf.at[pl.ds(tbase, N_PAD)], table)

        base0 = pl.multiple_of(j * G_PER, 8)
        pltpu.async_copy(rowi.at[pl.ds(base0, G_SUB)], ri0, sin0)
        pltpu.async_copy(coli.at[pl.ds(base0, G_SUB)], ci0, sin0)

        @pl.loop(0, n_sub // 2)
        def pair(g):
            for b in (0, 1):
                s = g * 2 + b
                base = j * G_PER + s * G_SUB

                @pl.when(s + 1 < n_sub)
                def _():
                    nbase = j * G_PER + (s + 1) * G_SUB
                    pltpu.async_copy(rowi.at[pl.ds(nbase, G_SUB)],
                                     ris[1 - b], sins[1 - b])
                    pltpu.async_copy(coli.at[pl.ds(nbase, G_SUB)],
                                     cis[1 - b], sins[1 - b])

                pltpu.make_async_copy(rowi.at[pl.ds(0, G_SUB)],
                                      ris[b], sins[b]).wait()
                pltpu.make_async_copy(coli.at[pl.ds(0, G_SUB)],
                                      cis[b], sins[b]).wait()

                @pl.when(s >= 2)
                def _():
                    pltpu.make_async_copy(difs[b],
                                          out.at[pl.ds(0, G_SUB)],
                                          souts[b]).wait()

                rb, cb, db = ris[b], cis[b], difs[b]

                @plsc.parallel_loop(0, G_SUB // LANES, unroll=4)
                def gath(i):
                    off = i * LANES
                    idr = rb[pl.ds(off, LANES)]
                    idc = cb[pl.ds(off, LANES)]
                    vr = plsc.load_gather(table, [idr])
                    vc = plsc.load_gather(table, [idc])
                    db[pl.ds(off, LANES)] = vr - vc

                obase = pl.multiple_of(comp * N_EDGES + base, 8)
                pltpu.async_copy(db, out.at[pl.ds(obase, G_SUB)], souts[b])

        for b in (0, 1):
            pltpu.make_async_copy(difs[b], out.at[pl.ds(0, G_SUB)],
                                  souts[b]).wait()


SW = N_PAD // NS


@functools.partial(
    pl.kernel,
    out_type=jax.ShapeDtypeStruct((NC * DIM * N_PAD,), jnp.float32),
    mesh=_mesh,
    compiler_params=pltpu.CompilerParams(needs_layout_passes=False),
    scratch_types=[
        pltpu.VMEM_SHARED((N_PAD,), jnp.float32),
        pltpu.VMEM_SHARED((N_PAD,), jnp.float32),
        pltpu.VMEM_SHARED((N_PAD,), jnp.float32),
        pltpu.VMEM((SW,), jnp.float32),
        pltpu.VMEM((S_SUB,), jnp.int32),
        pltpu.VMEM((S_SUB,), jnp.int32),
        pltpu.VMEM((S_SUB,), jnp.float32),
        pltpu.VMEM((S_SUB,), jnp.float32),
        pltpu.SemaphoreType.DMA,
        pltpu.SemaphoreType.DMA,
        pltpu.SemaphoreType.DMA,
        pltpu.SemaphoreType.DMA,
    ],
)
def _sc_scatter(emTf, rowi, out, acc0, acc1, acc2, zbuf,
                ri0, ri1, ev0, ev1, sri0, sri1, sev0, sev1):
    accs = (acc0, acc1, acc2)
    ris = (ri0, ri1)
    evs = (ev0, ev1)
    sris = (sri0, sri1)
    sevs = (sev0, sev1)
    cc = lax.axis_index("c")
    sid = lax.axis_index("s")
    n_sub = S_PER // S_SUB
    tile_base = (cc * NS + sid) * S_PER

    @plsc.parallel_loop(0, SW // LANES, unroll=8)
    def zfill(i):
        zbuf[pl.ds(i * LANES, LANES)] = jnp.zeros((LANES,), jnp.float32)

    for c in range(DIM):
        pltpu.sync_copy(zbuf, accs[c].at[pl.ds(sid * SW, SW)])
    plsc.subcore_barrier()

    def ev_src(k):
        s2, c2 = divmod(k, DIM)
        base2 = pl.multiple_of(c2 * N_EDGES + tile_base + s2 * S_SUB, 8)
        return emTf.at[pl.ds(base2, S_SUB)]

    pltpu.async_copy(rowi.at[pl.ds(tile_base, S_SUB)], ri0, sri0)
    pltpu.async_copy(ev_src(0), ev0, sev0)

    for s in range(n_sub):
        pltpu.make_async_copy(rowi.at[pl.ds(0, S_SUB)],
                              ris[s % 2], sris[s % 2]).wait()
        if s + 1 < n_sub:
            nb = pl.multiple_of(tile_base + (s + 1) * S_SUB, 8)
            pltpu.async_copy(rowi.at[pl.ds(nb, S_SUB)],
                             ris[(s + 1) % 2], sris[(s + 1) % 2])
        for c in range(DIM):
            k = s * DIM + c
            pltpu.make_async_copy(emTf.at[pl.ds(0, S_SUB)],
                                  evs[k % 2], sevs[k % 2]).wait()
            if k + 1 < n_sub * DIM:
                pltpu.async_copy(ev_src(k + 1),
                                 evs[(k + 1) % 2], sevs[(k + 1) % 2])
            pltpu.sync_copy(evs[k % 2], accs[c].at[ris[s % 2]], add=True)

    plsc.subcore_barrier()
    for c in range(DIM):
        abase = pl.multiple_of(sid * SW, 8)
        obase = pl.multiple_of(cc * ACC_FLAT + c * N_PAD + sid * SW, 8)
        pltpu.sync_copy(accs[c].at[pl.ds(abase, SW)], zbuf)
        pltpu.sync_copy(zbuf, out.at[pl.ds(obase, SW)])


@functools.partial(
    pl.kernel,
    out_type=jax.ShapeDtypeStruct((NC * N_PAD,), jnp.float32),
    mesh=_mesh,
    compiler_params=pltpu.CompilerParams(needs_layout_passes=False),
    scratch_types=[
        pltpu.VMEM_SHARED((CACC_FLAT,), jnp.float32),
        pltpu.VMEM((CACC_CHUNK,), jnp.float32),
        pltpu.VMEM((S_SUB,), jnp.int32),
        pltpu.VMEM((S_SUB,), jnp.float32),
    ],
)
def _sc_count(rowi, out, acc, zbuf, ri, ones):
    cc = lax.axis_index("c")
    sid = lax.axis_index("s")

    def zfill(i, c2):
        zbuf[pl.ds(i * LANES, LANES)] = jnp.zeros((LANES,), jnp.float32)
        return c2

    lax.fori_loop(0, CACC_CHUNK // LANES, zfill, 0)
    pltpu.sync_copy(zbuf, acc.at[pl.ds(sid * CACC_CHUNK, CACC_CHUNK)])

    def ones_body(i, c2):
        ones[pl.ds(i * LANES, LANES)] = jnp.ones((LANES,), jnp.float32)
        return c2

    lax.fori_loop(0, S_SUB // LANES, ones_body, 0)
    plsc.subcore_barrier()

    def sub_body(s, carry):
        base = (cc * NS + sid) * S_PER + s * S_SUB
        pltpu.sync_copy(rowi.at[pl.ds(base, S_SUB)], ri)
        pltpu.sync_copy(ones, acc.at[ri], add=True)
        return carry

    lax.fori_loop(0, S_PER // S_SUB, sub_body, 0)
    plsc.subcore_barrier()

    abase = pl.multiple_of(sid * CACC_CHUNK, 8)
    cbase = pl.multiple_of(cc * N_PAD + sid * CACC_CHUNK, 8)
    pltpu.sync_copy(acc.at[pl.ds(abase, CACC_CHUNK)], zbuf)
    pltpu.sync_copy(zbuf, out.at[pl.ds(cbase, CACC_CHUNK)])


def _edge_mlp_body(xd_ref, ea5_ref, w1e_ref, w1r_ref, w2_ref, out_ref):
    xd = xd_ref[...]
    radial = jnp.sqrt(jnp.sum(xd * xd, axis=0, keepdims=True))
    a = lax.dot_general(w1e_ref[...], ea5_ref[...], (((0,), (0,)), ((), ())),
                        preferred_element_type=jnp.float32)
    h = a + w1r_ref[...] * radial
    h = h * jax.nn.sigmoid(h)
    eo = lax.dot_general(w2_ref[...], h, (((0,), (0,)), ((), ())),
                         preferred_element_type=jnp.float32)
    out_ref[...] = xd * jnp.tanh(eo)


def _edge_mlp(xd, ea5, w1e, w1r, w2):
    grid = N_EDGES // EB
    return pl.pallas_call(
        _edge_mlp_body,
        grid=(grid,),
        in_specs=[
            pl.BlockSpec((DIM, EB), lambda i: (0, i)),
            pl.BlockSpec((1 + EA, EB), lambda i: (0, i)),
            pl.BlockSpec((1 + EA, NF), lambda i: (0, 0)),
            pl.BlockSpec((NF, 1), lambda i: (0, 0)),
            pl.BlockSpec((NF, 1), lambda i: (0, 0)),
        ],
        out_specs=pl.BlockSpec((DIM, EB), lambda i: (0, i)),
        out_shape=jax.ShapeDtypeStruct((DIM, N_EDGES), jnp.float32),
    )(xd, ea5, w1e, w1r, w2)


def _inv_cnt_body(part_ref, out_ref):
    p = part_ref[...]
    s = jnp.sum(p, axis=0, keepdims=True)
    out_ref[...] = 1.0 / jnp.maximum(s, 1.0)


def _inv_cnt(part):
    grid = N_PAD // NB
    return pl.pallas_call(
        _inv_cnt_body,
        grid=(grid,),
        in_specs=[pl.BlockSpec((NC, NB), lambda i: (0, i))],
        out_specs=pl.BlockSpec((1, NB), lambda i: (0, i)),
        out_shape=jax.ShapeDtypeStruct((1, N_PAD), jnp.float32),
    )(part)


def _update_body(xT_ref, part_ref, inv_ref, velT_ref, vn_ref,
                 vw1_ref, vb1_ref, vw2_ref, vb2_ref, out_ref):
    part = part_ref[...]
    agg = (part[0] + part[1]) * inv_ref[...]

    vn = vn_ref[...]
    h = vw1_ref[...][0][:, None] * vn + vb1_ref[...]
    h = h * jax.nn.sigmoid(h)
    vs = jnp.sum(h * vw2_ref[...], axis=0, keepdims=True) + vb2_ref[...]

    out_ref[...] = xT_ref[...] + agg + velT_ref[...] * vs


def _update(xT, part, inv, velT, vnT, vw1, vb1, vw2, vb2):
    grid = N_PAD // NB
    return pl.pallas_call(
        _update_body,
        grid=(grid,),
        in_specs=[
            pl.BlockSpec((DIM, NB), lambda i: (0, i)),
            pl.BlockSpec((NC, DIM, NB), lambda i: (0, 0, i)),
            pl.BlockSpec((1, NB), lambda i: (0, i)),
            pl.BlockSpec((DIM, NB), lambda i: (0, i)),
            pl.BlockSpec((1, NB), lambda i: (0, i)),
            pl.BlockSpec((1, NF), lambda i: (0, 0)),
            pl.BlockSpec((NF, 1), lambda i: (0, 0)),
            pl.BlockSpec((NF, 1), lambda i: (0, 0)),
            pl.BlockSpec((1, 1), lambda i: (0, 0)),
        ],
        out_specs=pl.BlockSpec((DIM, NB), lambda i: (0, i)),
        out_shape=jax.ShapeDtypeStruct((DIM, N_PAD), jnp.float32),
    )(xT, part, inv, velT, vnT, vw1, vb1, vw2, vb2)


def kernel(vel_norm, x, edges, vel, edge_attr, params):
    row = edges[0]
    col = edges[1]

    pad = N_PAD - N_NODES
    xT = jnp.pad(x.T, ((0, 0), (0, pad)))
    velT = jnp.pad(vel.T, ((0, 0), (0, pad)))
    vnT = jnp.pad(vel_norm.T, ((0, 0), (0, pad)))
    ea5 = jnp.concatenate(
        [edge_attr.T, jnp.ones((1, N_EDGES), jnp.float32)], axis=0)

    cnt_part = _sc_count(row).reshape(NC, N_PAD)
    inv = _inv_cnt(cnt_part)

    for p in params:
        w1e = jnp.concatenate([p['pW1'][1:], p['pb1'][None, :]], axis=0)
        w1r = p['pW1'][0].reshape(NF, 1)
        xd = _sc_gather_diff(xT.reshape(-1), row, col).reshape(DIM, N_EDGES)
        em = _edge_mlp(xd, ea5, w1e, w1r, p['pW2'])
        part = _sc_scatter(em.reshape(-1), row).reshape(NC, DIM, N_PAD)
        xT = _update(xT, part, inv, velT, vnT,
                     p['vW1'], p['vb1'].reshape(NF, 1), p['vW2'],
                     p['vb2'].reshape(1, 1))

    return xT[:, :N_NODES].T


# speedup vs baseline: 31.4446x; 2.9601x over previous
import functools

import jax
import jax.numpy as jnp
from jax import lax
from jax.experimental import pallas as pl
from jax.experimental.pallas import tpu as pltpu
from jax.experimental.pallas import tpu_sc as plsc

NC = 2
NS = 16
NW = NC * NS
LANES = 16

DIM = 3
N_NODES = 100000
N_PAD = 102400
N_EDGES = 1600000
NF = 32
EA = 4
NL = 4
EB = 5120
NB = 5120
E_PAD = 1602560
NEB = E_PAD // EB
NNB = N_PAD // NB

G_CHUNKS = 10
G_PER = N_EDGES // G_CHUNKS
G_SUB = 4000
S_PER = N_EDGES // NW
S_SUB = 10000
ACC_FLAT = DIM * N_PAD
ACC_CHUNK = ACC_FLAT // NS
CACC_FLAT = N_PAD
CACC_CHUNK = CACC_FLAT // NS

_mesh = plsc.VectorSubcoreMesh(core_axis_name="c", subcore_axis_name="s")


def _wid():
    return lax.axis_index("s") * NC + lax.axis_index("c")


@functools.partial(
    pl.kernel,
    out_type=jax.ShapeDtypeStruct((DIM * E_PAD,), jnp.float32),
    mesh=_mesh,
    compiler_params=pltpu.CompilerParams(needs_layout_passes=False),
    scratch_types=[
        pltpu.VMEM((N_PAD,), jnp.float32),
        pltpu.VMEM((G_SUB,), jnp.int32),
        pltpu.VMEM((G_SUB,), jnp.int32),
        pltpu.VMEM((G_SUB,), jnp.int32),
        pltpu.VMEM((G_SUB,), jnp.int32),
        pltpu.VMEM((G_SUB,), jnp.float32),
        pltpu.VMEM((G_SUB,), jnp.float32),
        pltpu.SemaphoreType.DMA,
        pltpu.SemaphoreType.DMA,
        pltpu.SemaphoreType.DMA,
        pltpu.SemaphoreType.DMA,
    ],
)
def _sc_gather_diff(xTf, rowi, coli, out, table, ri0, ri1, ci0, ci1,
                    dif0, dif1, sin0, sin1, sout0, sout1):
    wid = _wid()
    comp = wid // G_CHUNKS
    j = wid % G_CHUNKS
    n_sub = G_PER // G_SUB
    ris = (ri0, ri1)
    cis = (ci0, ci1)
    difs = (dif0, dif1)
    sins = (sin0, sin1)
    souts = (sout0, sout1)

    @pl.when(comp < DIM)
    def _():
        tbase = pl.multiple_of(comp * N_PAD, 8)
        pltpu.sync_copy(xTf.at[pl.ds(tbase, N_PAD)], table)

        base0 = pl.multiple_of(j * G_PER, 8)
        pltpu.async_copy(rowi.at[pl.ds(base0, G_SUB)], ri0, sin0)
        pltpu.async_copy(coli.at[pl.ds(base0, G_SUB)], ci0, sin0)

        @pl.loop(0, n_sub // 2)
        def pair(g):
            for b in (0, 1):
                s = g * 2 + b
                base = j * G_PER + s * G_SUB

                @pl.when(s + 1 < n_sub)
                def _():
                    nbase = j * G_PER + (s + 1) * G_SUB
                    pltpu.async_copy(rowi.at[pl.ds(nbase, G_SUB)],
                                     ris[1 - b], sins[1 - b])
                    pltpu.async_copy(coli.at[pl.ds(nbase, G_SUB)],
                                     cis[1 - b], sins[1 - b])

                pltpu.make_async_copy(rowi.at[pl.ds(0, G_SUB)],
                                      ris[b], sins[b]).wait()
                pltpu.make_async_copy(coli.at[pl.ds(0, G_SUB)],
                                      cis[b], sins[b]).wait()

                @pl.when(s >= 2)
                def _():
                    pltpu.make_async_copy(difs[b],
                                          out.at[pl.ds(0, G_SUB)],
                                          souts[b]).wait()

                rb, cb, db = ris[b], cis[b], difs[b]

                @plsc.parallel_loop(0, G_SUB // LANES, unroll=4)
                def gath(i):
                    off = i * LANES
                    idr = rb[pl.ds(off, LANES)]
                    idc = cb[pl.ds(off, LANES)]
                    vr = plsc.load_gather(table, [idr])
                    vc = plsc.load_gather(table, [idc])
                    db[pl.ds(off, LANES)] = vr - vc

                obase = pl.multiple_of(comp * E_PAD + base, 8)
                pltpu.async_copy(db, out.at[pl.ds(obase, G_SUB)], souts[b])

        for b in (0, 1):
            pltpu.make_async_copy(difs[b], out.at[pl.ds(0, G_SUB)],
                                  souts[b]).wait()


SW = N_PAD // NS


@functools.partial(
    pl.kernel,
    out_type=jax.ShapeDtypeStruct((NC * DIM * N_PAD,), jnp.float32),
    mesh=_mesh,
    compiler_params=pltpu.CompilerParams(needs_layout_passes=False),
    scratch_types=[
        pltpu.VMEM_SHARED((N_PAD,), jnp.float32),
        pltpu.VMEM_SHARED((N_PAD,), jnp.float32),
        pltpu.VMEM_SHARED((N_PAD,), jnp.float32),
        pltpu.VMEM((SW,), jnp.float32),
        pltpu.VMEM((S_SUB,), jnp.int32),
        pltpu.VMEM((S_SUB,), jnp.int32),
        pltpu.VMEM((S_SUB,), jnp.float32),
        pltpu.VMEM((S_SUB,), jnp.float32),
        pltpu.SemaphoreType.DMA,
        pltpu.SemaphoreType.DMA,
        pltpu.SemaphoreType.DMA,
        pltpu.SemaphoreType.DMA,
    ],
)
def _sc_scatter(em0, em1, em2, rowi, out, acc0, acc1, acc2, zbuf,
                ri0, ri1, ev0, ev1, sri0, sri1, sev0, sev1):
    accs = (acc0, acc1, acc2)
    ems = (em0, em1, em2)
    ris = (ri0, ri1)
    evs = (ev0, ev1)
    sris = (sri0, sri1)
    sevs = (sev0, sev1)
    cc = lax.axis_index("c")
    sid = lax.axis_index("s")
    n_sub = S_PER // S_SUB
    tile_base = (cc * NS + sid) * S_PER

    @plsc.parallel_loop(0, SW // LANES, unroll=8)
    def zfill(i):
        zbuf[pl.ds(i * LANES, LANES)] = jnp.zeros((LANES,), jnp.float32)

    for c in range(DIM):
        pltpu.sync_copy(zbuf, accs[c].at[pl.ds(sid * SW, SW)])
    plsc.subcore_barrier()

    def ev_src(k):
        s2, c2 = divmod(k, DIM)
        base2 = pl.multiple_of(tile_base + s2 * S_SUB, 8)
        return ems[c2].at[pl.ds(base2, S_SUB)]

    pltpu.async_copy(rowi.at[pl.ds(tile_base, S_SUB)], ri0, sri0)
    pltpu.async_copy(ev_src(0), ev0, sev0)

    for s in range(n_sub):
        pltpu.make_async_copy(rowi.at[pl.ds(0, S_SUB)],
                              ris[s % 2], sris[s % 2]).wait()
        if s + 1 < n_sub:
            nb = pl.multiple_of(tile_base + (s + 1) * S_SUB, 8)
            pltpu.async_copy(rowi.at[pl.ds(nb, S_SUB)],
                             ris[(s + 1) % 2], sris[(s + 1) % 2])
        for c in range(DIM):
            k = s * DIM + c
            pltpu.make_async_copy(em0.at[pl.ds(0, S_SUB)],
                                  evs[k % 2], sevs[k % 2]).wait()
            if k + 1 < n_sub * DIM:
                pltpu.async_copy(ev_src(k + 1),
                                 evs[(k + 1) % 2], sevs[(k + 1) % 2])
            pltpu.sync_copy(evs[k % 2], accs[c].at[ris[s % 2]], add=True)

    plsc.subcore_barrier()
    for c in range(DIM):
        abase = pl.multiple_of(sid * SW, 8)
        obase = pl.multiple_of(cc * ACC_FLAT + c * N_PAD + sid * SW, 8)
        pltpu.sync_copy(accs[c].at[pl.ds(abase, SW)], zbuf)
        pltpu.sync_copy(zbuf, out.at[pl.ds(obase, SW)])


@functools.partial(
    pl.kernel,
    out_type=jax.ShapeDtypeStruct((NC * N_PAD,), jnp.float32),
    mesh=_mesh,
    compiler_params=pltpu.CompilerParams(needs_layout_passes=False),
    scratch_types=[
        pltpu.VMEM_SHARED((CACC_FLAT,), jnp.float32),
        pltpu.VMEM((CACC_CHUNK,), jnp.float32),
        pltpu.VMEM((S_SUB,), jnp.int32),
        pltpu.VMEM((S_SUB,), jnp.float32),
    ],
)
def _sc_count(rowi, out, acc, zbuf, ri, ones):
    cc = lax.axis_index("c")
    sid = lax.axis_index("s")

    def zfill(i, c2):
        zbuf[pl.ds(i * LANES, LANES)] = jnp.zeros((LANES,), jnp.float32)
        return c2

    lax.fori_loop(0, CACC_CHUNK // LANES, zfill, 0)
    pltpu.sync_copy(zbuf, acc.at[pl.ds(sid * CACC_CHUNK, CACC_CHUNK)])

    def ones_body(i, c2):
        ones[pl.ds(i * LANES, LANES)] = jnp.ones((LANES,), jnp.float32)
        return c2

    lax.fori_loop(0, S_SUB // LANES, ones_body, 0)
    plsc.subcore_barrier()

    def sub_body(s, carry):
        base = (cc * NS + sid) * S_PER + s * S_SUB
        pltpu.sync_copy(rowi.at[pl.ds(base, S_SUB)], ri)
        pltpu.sync_copy(ones, acc.at[ri], add=True)
        return carry

    lax.fori_loop(0, S_PER // S_SUB, sub_body, 0)
    plsc.subcore_barrier()

    abase = pl.multiple_of(sid * CACC_CHUNK, 8)
    cbase = pl.multiple_of(cc * N_PAD + sid * CACC_CHUNK, 8)
    pltpu.sync_copy(acc.at[pl.ds(abase, CACC_CHUNK)], zbuf)
    pltpu.sync_copy(zbuf, out.at[pl.ds(cbase, CACC_CHUNK)])


def _edge_mlp_body(xd0_ref, xd1_ref, xd2_ref,
                   ea0_ref, ea1_ref, ea2_ref, ea3_ref,
                   w1a_ref, w2_ref, o0_ref, o1_ref, o2_ref):
    xd0 = xd0_ref[...]
    xd1 = xd1_ref[...]
    xd2 = xd2_ref[...]
    r = jnp.sqrt(xd0 * xd0 + xd1 * xd1 + xd2 * xd2)
    e_in = jnp.stack([r, ea0_ref[...], ea1_ref[...], ea2_ref[...],
                      ea3_ref[...], jnp.ones_like(r)])
    h = lax.dot_general(w1a_ref[...], e_in, (((0,), (0,)), ((), ())),
                        preferred_element_type=jnp.float32)
    h = h * jax.nn.sigmoid(h)
    eo = lax.dot_general(w2_ref[...], h, (((0,), (0,)), ((), ())),
                         preferred_element_type=jnp.float32)
    t = jnp.tanh(eo)[0]
    o0_ref[...] = xd0 * t
    o1_ref[...] = xd1 * t
    o2_ref[...] = xd2 * t


def _edge_mlp(xdf, eaf, w1a, w2):
    espec = jax.ShapeDtypeStruct((E_PAD,), jnp.float32)
    return pl.pallas_call(
        _edge_mlp_body,
        grid=(NEB,),
        in_specs=[
            pl.BlockSpec((EB,), lambda i: (i,)),
            pl.BlockSpec((EB,), lambda i: (NEB + i,)),
            pl.BlockSpec((EB,), lambda i: (2 * NEB + i,)),
            pl.BlockSpec((EB,), lambda i: (i,)),
            pl.BlockSpec((EB,), lambda i: (NEB + i,)),
            pl.BlockSpec((EB,), lambda i: (2 * NEB + i,)),
            pl.BlockSpec((EB,), lambda i: (3 * NEB + i,)),
            pl.BlockSpec((1 + EA + 1, NF), lambda i: (0, 0)),
            pl.BlockSpec((NF, 1), lambda i: (0, 0)),
        ],
        out_specs=(
            pl.BlockSpec((EB,), lambda i: (i,)),
            pl.BlockSpec((EB,), lambda i: (i,)),
            pl.BlockSpec((EB,), lambda i: (i,)),
        ),
        out_shape=(espec, espec, espec),
    )(xdf, xdf, xdf, eaf, eaf, eaf, eaf, w1a, w2)


def _inv_cnt_body(p0_ref, p1_ref, out_ref):
    s = p0_ref[...] + p1_ref[...]
    out_ref[...] = 1.0 / jnp.maximum(s, 1.0)


def _inv_cnt(cntf):
    return pl.pallas_call(
        _inv_cnt_body,
        grid=(NNB,),
        in_specs=[
            pl.BlockSpec((NB,), lambda i: (i,)),
            pl.BlockSpec((NB,), lambda i: (NNB + i,)),
        ],
        out_specs=pl.BlockSpec((NB,), lambda i: (i,)),
        out_shape=jax.ShapeDtypeStruct((N_PAD,), jnp.float32),
    )(cntf, cntf)


def _vscale_body(vn_ref, w1_ref, b1_ref, w2_ref, b2_ref, out_ref):
    vn = vn_ref[...][None, :]
    w1 = w1_ref[...]
    b1 = b1_ref[...]
    w2 = w2_ref[...]
    b2 = b2_ref[...]
    rows = []
    for l in range(NL):
        h = w1[l][:, None] * vn + b1[l][:, None]
        h = h * jax.nn.sigmoid(h)
        rows.append(jnp.sum(h * w2[l][:, None], axis=0) + b2[l, 0])
    out_ref[...] = jnp.stack(rows)


def _vscale(vnf, w1s, b1s, w2s, b2s):
    return pl.pallas_call(
        _vscale_body,
        grid=(NNB,),
        in_specs=[
            pl.BlockSpec((NB,), lambda n: (n,)),
            pl.BlockSpec((NL, NF), lambda n: (0, 0)),
            pl.BlockSpec((NL, NF), lambda n: (0, 0)),
            pl.BlockSpec((NL, NF), lambda n: (0, 0)),
            pl.BlockSpec((NL, 1), lambda n: (0, 0)),
        ],
        out_specs=pl.BlockSpec((NL, NB), lambda n: (0, n)),
        out_shape=jax.ShapeDtypeStruct((NL, N_PAD), jnp.float32),
    )(vnf, w1s, b1s, w2s, b2s)


def _update_body(layer, x_ref, p0_ref, p1_ref, inv_ref, vel_ref, vs_ref,
                 out_ref):
    agg = (p0_ref[...] + p1_ref[...]) * inv_ref[...]
    vs = vs_ref[...][layer]
    out_ref[...] = x_ref[...] + agg + vel_ref[...] * vs


def _update(xTf, partf, invf, velTf, vsf, layer):
    nblk = DIM * NNB
    return pl.pallas_call(
        functools.partial(_update_body, layer),
        grid=(nblk,),
        in_specs=[
            pl.BlockSpec((NB,), lambda i: (i,)),
            pl.BlockSpec((NB,), lambda i: (i,)),
            pl.BlockSpec((NB,), lambda i: (nblk + i,)),
            pl.BlockSpec((NB,), lambda i: (i % NNB,)),
            pl.BlockSpec((NB,), lambda i: (i,)),
            pl.BlockSpec((NL, NB), lambda i: (0, i % NNB)),
        ],
        out_specs=pl.BlockSpec((NB,), lambda i: (i,)),
        out_shape=jax.ShapeDtypeStruct((DIM * N_PAD,), jnp.float32),
    )(xTf, partf, partf, invf, velTf, vsf)


def kernel(vel_norm, x, edges, vel, edge_attr, params):
    row = edges[0]
    col = edges[1]

    pad = N_PAD - N_NODES
    xTf = jnp.pad(x.T, ((0, 0), (0, pad))).reshape(-1)
    velTf = jnp.pad(vel.T, ((0, 0), (0, pad))).reshape(-1)
    vnf = jnp.pad(vel_norm[:, 0], (0, pad))
    eaf = jnp.pad(edge_attr.T, ((0, 0), (0, E_PAD - N_EDGES))).reshape(-1)

    cntf = _sc_count(row)
    invf = _inv_cnt(cntf)

    w1s = jnp.stack([p['vW1'][0] for p in params])
    b1s = jnp.stack([p['vb1'] for p in params])
    w2s = jnp.stack([p['vW2'][:, 0] for p in params])
    b2s = jnp.stack([p['vb2'] for p in params])
    vsf = _vscale(vnf, w1s, b1s, w2s, b2s)

    for l, p in enumerate(params):
        w1a = jnp.concatenate([p['pW1'], p['pb1'][None, :]], axis=0)
        xdf = _sc_gather_diff(xTf, row, col)
        em0, em1, em2 = _edge_mlp(xdf, eaf, w1a, p['pW2'])
        partf = _sc_scatter(em0, em1, em2, row)
        xTf = _update(xTf, partf, invf, velTf, vsf, l)

    return xTf.reshape(DIM, N_PAD)[:, :N_NODES].T


# speedup vs baseline: 43.8857x; 1.3957x over previous
import functools

import jax
import jax.numpy as jnp
from jax import lax
from jax.experimental import pallas as pl
from jax.experimental.pallas import tpu as pltpu
from jax.experimental.pallas import tpu_sc as plsc

NC = 2
NS = 16
NW = NC * NS
LANES = 16

DIM = 3
N_NODES = 100000
N_PAD = 102400
N_EDGES = 1600000
NF = 32
EA = 4
NL = 4
EB = 10240
NB = 10240
E_PAD = 1607680
NEB = E_PAD // EB
NNB = N_PAD // NB

G_CHUNKS = 10
G_PER = N_EDGES // G_CHUNKS
G_SUB = 4000
S_PER = N_EDGES // NW
S_SUB = 10000
ACC_FLAT = DIM * N_PAD
ACC_CHUNK = ACC_FLAT // NS
CACC_FLAT = N_PAD
CACC_CHUNK = CACC_FLAT // NS

_mesh = plsc.VectorSubcoreMesh(core_axis_name="c", subcore_axis_name="s")


def _wid():
    return lax.axis_index("s") * NC + lax.axis_index("c")


@functools.partial(
    pl.kernel,
    out_type=jax.ShapeDtypeStruct((DIM * E_PAD,), jnp.float32),
    mesh=_mesh,
    compiler_params=pltpu.CompilerParams(needs_layout_passes=False),
    scratch_types=[
        pltpu.VMEM((N_PAD,), jnp.float32),
        pltpu.VMEM((G_SUB,), jnp.int32),
        pltpu.VMEM((G_SUB,), jnp.int32),
        pltpu.VMEM((G_SUB,), jnp.int32),
        pltpu.VMEM((G_SUB,), jnp.int32),
        pltpu.VMEM((G_SUB,), jnp.float32),
        pltpu.VMEM((G_SUB,), jnp.float32),
        pltpu.SemaphoreType.DMA,
        pltpu.SemaphoreType.DMA,
        pltpu.SemaphoreType.DMA,
        pltpu.SemaphoreType.DMA,
    ],
)
def _sc_gather_diff(xTf, rowi, coli, out, table, ri0, ri1, ci0, ci1,
                    dif0, dif1, sin0, sin1, sout0, sout1):
    wid = _wid()
    comp = wid // G_CHUNKS
    j = wid % G_CHUNKS
    n_sub = G_PER // G_SUB
    ris = (ri0, ri1)
    cis = (ci0, ci1)
    difs = (dif0, dif1)
    sins = (sin0, sin1)
    souts = (sout0, sout1)

    @pl.when(comp < DIM)
    def _():
        tbase = pl.multiple_of(comp * N_PAD, 8)
        pltpu.sync_copy(xTf.at[pl.ds(tbase, N_PAD)], table)

        base0 = pl.multiple_of(j * G_PER, 8)
        pltpu.async_copy(rowi.at[pl.ds(base0, G_SUB)], ri0, sin0)
        pltpu.async_copy(coli.at[pl.ds(base0, G_SUB)], ci0, sin0)

        @pl.loop(0, n_sub // 2)
        def pair(g):
            for b in (0, 1):
                s = g * 2 + b
                base = j * G_PER + s * G_SUB

                @pl.when(s + 1 < n_sub)
                def _():
                    nbase = j * G_PER + (s + 1) * G_SUB
                    pltpu.async_copy(rowi.at[pl.ds(nbase, G_SUB)],
                                     ris[1 - b], sins[1 - b])
                    pltpu.async_copy(coli.at[pl.ds(nbase, G_SUB)],
                                     cis[1 - b], sins[1 - b])

                pltpu.make_async_copy(rowi.at[pl.ds(0, G_SUB)],
                                      ris[b], sins[b]).wait()
                pltpu.make_async_copy(coli.at[pl.ds(0, G_SUB)],
                                      cis[b], sins[b]).wait()

                @pl.when(s >= 2)
                def _():
                    pltpu.make_async_copy(difs[b],
                                          out.at[pl.ds(0, G_SUB)],
                                          souts[b]).wait()

                rb, cb, db = ris[b], cis[b], difs[b]

                @plsc.parallel_loop(0, G_SUB // LANES, unroll=4)
                def gath(i):
                    off = i * LANES
                    idr = rb[pl.ds(off, LANES)]
                    idc = cb[pl.ds(off, LANES)]
                    vr = plsc.load_gather(table, [idr])
                    vc = plsc.load_gather(table, [idc])
                    db[pl.ds(off, LANES)] = vr - vc

                obase = pl.multiple_of(comp * E_PAD + base, 8)
                pltpu.async_copy(db, out.at[pl.ds(obase, G_SUB)], souts[b])

        for b in (0, 1):
            pltpu.make_async_copy(difs[b], out.at[pl.ds(0, G_SUB)],
                                  souts[b]).wait()


SW = N_PAD // NS


@functools.partial(
    pl.kernel,
    out_type=jax.ShapeDtypeStruct((NC * DIM * N_PAD,), jnp.float32),
    mesh=_mesh,
    compiler_params=pltpu.CompilerParams(needs_layout_passes=False),
    scratch_types=[
        pltpu.VMEM_SHARED((N_PAD,), jnp.float32),
        pltpu.VMEM_SHARED((N_PAD,), jnp.float32),
        pltpu.VMEM_SHARED((N_PAD,), jnp.float32),
        pltpu.VMEM((SW,), jnp.float32),
        pltpu.VMEM((S_SUB,), jnp.int32),
        pltpu.VMEM((S_SUB,), jnp.int32),
        pltpu.VMEM((S_SUB,), jnp.float32),
        pltpu.VMEM((S_SUB,), jnp.float32),
        pltpu.SemaphoreType.DMA,
        pltpu.SemaphoreType.DMA,
        pltpu.SemaphoreType.DMA,
        pltpu.SemaphoreType.DMA,
    ],
)
def _sc_scatter(em0, em1, em2, rowi, out, acc0, acc1, acc2, zbuf,
                ri0, ri1, ev0, ev1, sri0, sri1, sev0, sev1):
    accs = (acc0, acc1, acc2)
    ems = (em0, em1, em2)
    ris = (ri0, ri1)
    evs = (ev0, ev1)
    sris = (sri0, sri1)
    sevs = (sev0, sev1)
    cc = lax.axis_index("c")
    sid = lax.axis_index("s")
    n_sub = S_PER // S_SUB
    tile_base = (cc * NS + sid) * S_PER

    @plsc.parallel_loop(0, SW // LANES, unroll=8)
    def zfill(i):
        zbuf[pl.ds(i * LANES, LANES)] = jnp.zeros((LANES,), jnp.float32)

    for c in range(DIM):
        pltpu.sync_copy(zbuf, accs[c].at[pl.ds(sid * SW, SW)])
    plsc.subcore_barrier()

    def ev_src(k):
        s2, c2 = divmod(k, DIM)
        base2 = pl.multiple_of(tile_base + s2 * S_SUB, 8)
        return ems[c2].at[pl.ds(base2, S_SUB)]

    pltpu.async_copy(rowi.at[pl.ds(tile_base, S_SUB)], ri0, sri0)
    pltpu.async_copy(ev_src(0), ev0, sev0)

    for s in range(n_sub):
        pltpu.make_async_copy(rowi.at[pl.ds(0, S_SUB)],
                              ris[s % 2], sris[s % 2]).wait()
        if s + 1 < n_sub:
            nb = pl.multiple_of(tile_base + (s + 1) * S_SUB, 8)
            pltpu.async_copy(rowi.at[pl.ds(nb, S_SUB)],
                             ris[(s + 1) % 2], sris[(s + 1) % 2])
        for c in range(DIM):
            k = s * DIM + c
            pltpu.make_async_copy(em0.at[pl.ds(0, S_SUB)],
                                  evs[k % 2], sevs[k % 2]).wait()
            if k + 1 < n_sub * DIM:
                pltpu.async_copy(ev_src(k + 1),
                                 evs[(k + 1) % 2], sevs[(k + 1) % 2])
            pltpu.sync_copy(evs[k % 2], accs[c].at[ris[s % 2]], add=True)

    plsc.subcore_barrier()
    for c in range(DIM):
        abase = pl.multiple_of(sid * SW, 8)
        obase = pl.multiple_of(cc * ACC_FLAT + c * N_PAD + sid * SW, 8)
        pltpu.sync_copy(accs[c].at[pl.ds(abase, SW)], zbuf)
        pltpu.sync_copy(zbuf, out.at[pl.ds(obase, SW)])


@functools.partial(
    pl.kernel,
    out_type=jax.ShapeDtypeStruct((NC * N_PAD,), jnp.float32),
    mesh=_mesh,
    compiler_params=pltpu.CompilerParams(needs_layout_passes=False),
    scratch_types=[
        pltpu.VMEM_SHARED((CACC_FLAT,), jnp.float32),
        pltpu.VMEM((CACC_CHUNK,), jnp.float32),
        pltpu.VMEM((S_SUB,), jnp.int32),
        pltpu.VMEM((S_SUB,), jnp.float32),
    ],
)
def _sc_count(rowi, out, acc, zbuf, ri, ones):
    cc = lax.axis_index("c")
    sid = lax.axis_index("s")

    def zfill(i, c2):
        zbuf[pl.ds(i * LANES, LANES)] = jnp.zeros((LANES,), jnp.float32)
        return c2

    lax.fori_loop(0, CACC_CHUNK // LANES, zfill, 0)
    pltpu.sync_copy(zbuf, acc.at[pl.ds(sid * CACC_CHUNK, CACC_CHUNK)])

    def ones_body(i, c2):
        ones[pl.ds(i * LANES, LANES)] = jnp.ones((LANES,), jnp.float32)
        return c2

    lax.fori_loop(0, S_SUB // LANES, ones_body, 0)
    plsc.subcore_barrier()

    def sub_body(s, carry):
        base = (cc * NS + sid) * S_PER + s * S_SUB
        pltpu.sync_copy(rowi.at[pl.ds(base, S_SUB)], ri)
        pltpu.sync_copy(ones, acc.at[ri], add=True)
        return carry

    lax.fori_loop(0, S_PER // S_SUB, sub_body, 0)
    plsc.subcore_barrier()

    abase = pl.multiple_of(sid * CACC_CHUNK, 8)
    cbase = pl.multiple_of(cc * N_PAD + sid * CACC_CHUNK, 8)
    pltpu.sync_copy(acc.at[pl.ds(abase, CACC_CHUNK)], zbuf)
    pltpu.sync_copy(zbuf, out.at[pl.ds(cbase, CACC_CHUNK)])


def _edge_mlp_body(xd0_ref, xd1_ref, xd2_ref,
                   ea0_ref, ea1_ref, ea2_ref, ea3_ref,
                   w1a_ref, w2_ref, o0_ref, o1_ref, o2_ref):
    xd0 = xd0_ref[...]
    xd1 = xd1_ref[...]
    xd2 = xd2_ref[...]
    r = jnp.sqrt(xd0 * xd0 + xd1 * xd1 + xd2 * xd2)
    e_in = jnp.stack([r, ea0_ref[...], ea1_ref[...], ea2_ref[...],
                      ea3_ref[...], jnp.ones_like(r)])
    h = lax.dot_general(w1a_ref[...], e_in, (((0,), (0,)), ((), ())),
                        preferred_element_type=jnp.float32)
    h = h * jax.nn.sigmoid(h)
    eo = lax.dot_general(w2_ref[...], h, (((0,), (0,)), ((), ())),
                         preferred_element_type=jnp.float32)
    t = jnp.tanh(eo)[0]
    o0_ref[...] = xd0 * t
    o1_ref[...] = xd1 * t
    o2_ref[...] = xd2 * t


def _edge_mlp(xdf, eaf, w1a, w2):
    espec = jax.ShapeDtypeStruct((E_PAD,), jnp.float32)
    return pl.pallas_call(
        _edge_mlp_body,
        grid=(NEB,),
        in_specs=[
            pl.BlockSpec((EB,), lambda i: (i,)),
            pl.BlockSpec((EB,), lambda i: (NEB + i,)),
            pl.BlockSpec((EB,), lambda i: (2 * NEB + i,)),
            pl.BlockSpec((EB,), lambda i: (i,)),
            pl.BlockSpec((EB,), lambda i: (NEB + i,)),
            pl.BlockSpec((EB,), lambda i: (2 * NEB + i,)),
            pl.BlockSpec((EB,), lambda i: (3 * NEB + i,)),
            pl.BlockSpec((1 + EA + 1, NF), lambda i: (0, 0)),
            pl.BlockSpec((NF, 1), lambda i: (0, 0)),
        ],
        out_specs=(
            pl.BlockSpec((EB,), lambda i: (i,)),
            pl.BlockSpec((EB,), lambda i: (i,)),
            pl.BlockSpec((EB,), lambda i: (i,)),
        ),
        out_shape=(espec, espec, espec),
    )(xdf, xdf, xdf, eaf, eaf, eaf, eaf, w1a, w2)


def _inv_cnt_body(p0_ref, p1_ref, out_ref):
    s = p0_ref[...] + p1_ref[...]
    out_ref[...] = 1.0 / jnp.maximum(s, 1.0)


def _inv_cnt(cntf):
    return pl.pallas_call(
        _inv_cnt_body,
        grid=(NNB,),
        in_specs=[
            pl.BlockSpec((NB,), lambda i: (i,)),
            pl.BlockSpec((NB,), lambda i: (NNB + i,)),
        ],
        out_specs=pl.BlockSpec((NB,), lambda i: (i,)),
        out_shape=jax.ShapeDtypeStruct((N_PAD,), jnp.float32),
    )(cntf, cntf)


def _vscale_body(vn_ref, w1_ref, b1_ref, w2_ref, b2_ref, out_ref):
    vn = vn_ref[...][None, :]
    w1 = w1_ref[...]
    b1 = b1_ref[...]
    w2 = w2_ref[...]
    b2 = b2_ref[...]
    rows = []
    for l in range(NL):
        h = w1[l][:, None] * vn + b1[l][:, None]
        h = h * jax.nn.sigmoid(h)
        rows.append(jnp.sum(h * w2[l][:, None], axis=0) + b2[l, 0])
    out_ref[...] = jnp.stack(rows)


def _vscale(vnf, w1s, b1s, w2s, b2s):
    return pl.pallas_call(
        _vscale_body,
        grid=(NNB,),
        in_specs=[
            pl.BlockSpec((NB,), lambda n: (n,)),
            pl.BlockSpec((NL, NF), lambda n: (0, 0)),
            pl.BlockSpec((NL, NF), lambda n: (0, 0)),
            pl.BlockSpec((NL, NF), lambda n: (0, 0)),
            pl.BlockSpec((NL, 1), lambda n: (0, 0)),
        ],
        out_specs=pl.BlockSpec((NL, NB), lambda n: (0, n)),
        out_shape=jax.ShapeDtypeStruct((NL, N_PAD), jnp.float32),
    )(vnf, w1s, b1s, w2s, b2s)


def _update_body(layer, x_ref, p0_ref, p1_ref, inv_ref, vel_ref, vs_ref,
                 out_ref):
    agg = (p0_ref[...] + p1_ref[...]) * inv_ref[...]
    vs = vs_ref[...][layer]
    out_ref[...] = x_ref[...] + agg + vel_ref[...] * vs


def _update(xTf, partf, invf, velTf, vsf, layer):
    nblk = DIM * NNB
    return pl.pallas_call(
        functools.partial(_update_body, layer),
        grid=(nblk,),
        in_specs=[
            pl.BlockSpec((NB,), lambda i: (i,)),
            pl.BlockSpec((NB,), lambda i: (i,)),
            pl.BlockSpec((NB,), lambda i: (nblk + i,)),
            pl.BlockSpec((NB,), lambda i: (i % NNB,)),
            pl.BlockSpec((NB,), lambda i: (i,)),
            pl.BlockSpec((NL, NB), lambda i: (0, i % NNB)),
        ],
        out_specs=pl.BlockSpec((NB,), lambda i: (i,)),
        out_shape=jax.ShapeDtypeStruct((DIM * N_PAD,), jnp.float32),
    )(xTf, partf, partf, invf, velTf, vsf)


def kernel(vel_norm, x, edges, vel, edge_attr, params):
    row = edges[0]
    col = edges[1]

    pad = N_PAD - N_NODES
    xTf = jnp.pad(x.T, ((0, 0), (0, pad))).reshape(-1)
    velTf = jnp.pad(vel.T, ((0, 0), (0, pad))).reshape(-1)
    vnf = jnp.pad(vel_norm[:, 0], (0, pad))
    eaf = jnp.pad(edge_attr.T, ((0, 0), (0, E_PAD - N_EDGES))).reshape(-1)

    cntf = _sc_count(row)
    invf = _inv_cnt(cntf)

    w1s = jnp.stack([p['vW1'][0] for p in params])
    b1s = jnp.stack([p['vb1'] for p in params])
    w2s = jnp.stack([p['vW2'][:, 0] for p in params])
    b2s = jnp.stack([p['vb2'] for p in params])
    vsf = _vscale(vnf, w1s, b1s, w2s, b2s)

    for l, p in enumerate(params):
        w1a = jnp.concatenate([p['pW1'], p['pb1'][None, :]], axis=0)
        xdf = _sc_gather_diff(xTf, row, col)
        em0, em1, em2 = _edge_mlp(xdf, eaf, w1a, p['pW2'])
        partf = _sc_scatter(em0, em1, em2, row)
        xTf = _update(xTf, partf, invf, velTf, vsf, l)

    return xTf.reshape(DIM, N_PAD)[:, :N_NODES].T


# speedup vs baseline: 45.8447x; 1.0446x over previous
import functools

import jax
import jax.numpy as jnp
from jax import lax
from jax.experimental import pallas as pl
from jax.experimental.pallas import tpu as pltpu
from jax.experimental.pallas import tpu_sc as plsc

NC = 2
NS = 16
NW = NC * NS
LANES = 16

DIM = 3
N_NODES = 100000
N_PAD = 102400
N_EDGES = 1600000
NF = 32
EA = 4
NL = 4
EB = 10240
NB = 10240
E_PAD = 1607680
NEB = E_PAD // EB
NNB = N_PAD // NB

G_CHUNKS = 10
G_PER = N_EDGES // G_CHUNKS
G_SUB = 4000
S_PER = N_EDGES // NW
S_SUB = 10000
ACC_FLAT = DIM * N_PAD
ACC_CHUNK = ACC_FLAT // NS
CACC_FLAT = N_PAD
CACC_CHUNK = CACC_FLAT // NS

_mesh = plsc.VectorSubcoreMesh(core_axis_name="c", subcore_axis_name="s")


def _wid():
    return lax.axis_index("s") * NC + lax.axis_index("c")


@functools.partial(
    pl.kernel,
    out_type=jax.ShapeDtypeStruct((DIM * E_PAD,), jnp.float32),
    mesh=_mesh,
    compiler_params=pltpu.CompilerParams(needs_layout_passes=False),
    scratch_types=[
        pltpu.VMEM((N_PAD,), jnp.float32),
        pltpu.VMEM((G_SUB,), jnp.int32),
        pltpu.VMEM((G_SUB,), jnp.int32),
        pltpu.VMEM((G_SUB,), jnp.int32),
        pltpu.VMEM((G_SUB,), jnp.int32),
        pltpu.VMEM((G_SUB,), jnp.float32),
        pltpu.VMEM((G_SUB,), jnp.float32),
        pltpu.SemaphoreType.DMA,
        pltpu.SemaphoreType.DMA,
        pltpu.SemaphoreType.DMA,
        pltpu.SemaphoreType.DMA,
    ],
)
def _sc_gather_diff(xTf, rowi, coli, out, table, ri0, ri1, ci0, ci1,
                    dif0, dif1, sin0, sin1, sout0, sout1):
    wid = _wid()
    comp = wid // G_CHUNKS
    j = wid % G_CHUNKS
    n_sub = G_PER // G_SUB
    ris = (ri0, ri1)
    cis = (ci0, ci1)
    difs = (dif0, dif1)
    sins = (sin0, sin1)
    souts = (sout0, sout1)

    @pl.when(comp < DIM)
    def _():
        tbase = pl.multiple_of(comp * N_PAD, 8)
        pltpu.sync_copy(xTf.at[pl.ds(tbase, N_PAD)], table)

        base0 = pl.multiple_of(j * G_PER, 8)
        pltpu.async_copy(rowi.at[pl.ds(base0, G_SUB)], ri0, sin0)
        pltpu.async_copy(coli.at[pl.ds(base0, G_SUB)], ci0, sin0)

        @pl.loop(0, n_sub // 2)
        def pair(g):
            for b in (0, 1):
                s = g * 2 + b
                base = j * G_PER + s * G_SUB

                @pl.when(s + 1 < n_sub)
                def _():
                    nbase = j * G_PER + (s + 1) * G_SUB
                    pltpu.async_copy(rowi.at[pl.ds(nbase, G_SUB)],
                                     ris[1 - b], sins[1 - b])
                    pltpu.async_copy(coli.at[pl.ds(nbase, G_SUB)],
                                     cis[1 - b], sins[1 - b])

                pltpu.make_async_copy(rowi.at[pl.ds(0, G_SUB)],
                                      ris[b], sins[b]).wait()
                pltpu.make_async_copy(coli.at[pl.ds(0, G_SUB)],
                                      cis[b], sins[b]).wait()

                @pl.when(s >= 2)
                def _():
                    pltpu.make_async_copy(difs[b],
                                          out.at[pl.ds(0, G_SUB)],
                                          souts[b]).wait()

                rb, cb, db = ris[b], cis[b], difs[b]

                @plsc.parallel_loop(0, G_SUB // LANES, unroll=4)
                def gath(i):
                    off = i * LANES
                    idr = rb[pl.ds(off, LANES)]
                    idc = cb[pl.ds(off, LANES)]
                    vr = plsc.load_gather(table, [idr])
                    vc = plsc.load_gather(table, [idc])
                    db[pl.ds(off, LANES)] = vr - vc

                obase = pl.multiple_of(comp * E_PAD + base, 8)
                pltpu.async_copy(db, out.at[pl.ds(obase, G_SUB)], souts[b])

        for b in (0, 1):
            pltpu.make_async_copy(difs[b], out.at[pl.ds(0, G_SUB)],
                                  souts[b]).wait()


SW = N_PAD // NS


@functools.partial(
    pl.kernel,
    out_type=jax.ShapeDtypeStruct((NC * DIM * N_PAD,), jnp.float32),
    mesh=_mesh,
    compiler_params=pltpu.CompilerParams(needs_layout_passes=False),
    scratch_types=[
        pltpu.VMEM_SHARED((N_PAD,), jnp.float32),
        pltpu.VMEM_SHARED((N_PAD,), jnp.float32),
        pltpu.VMEM_SHARED((N_PAD,), jnp.float32),
        pltpu.VMEM((SW,), jnp.float32),
        pltpu.VMEM((S_SUB,), jnp.int32),
        pltpu.VMEM((S_SUB,), jnp.int32),
        pltpu.VMEM((S_SUB,), jnp.float32),
        pltpu.VMEM((S_SUB,), jnp.float32),
        pltpu.SemaphoreType.DMA,
        pltpu.SemaphoreType.DMA,
        pltpu.SemaphoreType.DMA,
        pltpu.SemaphoreType.DMA,
    ],
)
def _sc_scatter(em0, em1, em2, rowi, out, acc0, acc1, acc2, zbuf,
                ri0, ri1, ev0, ev1, sri0, sri1, sev0, sev1):
    accs = (acc0, acc1, acc2)
    ems = (em0, em1, em2)
    ris = (ri0, ri1)
    evs = (ev0, ev1)
    sris = (sri0, sri1)
    sevs = (sev0, sev1)
    cc = lax.axis_index("c")
    sid = lax.axis_index("s")
    n_sub = S_PER // S_SUB
    tile_base = (cc * NS + sid) * S_PER

    @plsc.parallel_loop(0, SW // LANES, unroll=8)
    def zfill(i):
        zbuf[pl.ds(i * LANES, LANES)] = jnp.zeros((LANES,), jnp.float32)

    for c in range(DIM):
        pltpu.sync_copy(zbuf, accs[c].at[pl.ds(sid * SW, SW)])
    plsc.subcore_barrier()

    def ev_src(k):
        s2, c2 = divmod(k, DIM)
        base2 = pl.multiple_of(tile_base + s2 * S_SUB, 8)
        return ems[c2].at[pl.ds(base2, S_SUB)]

    pltpu.async_copy(rowi.at[pl.ds(tile_base, S_SUB)], ri0, sri0)
    pltpu.async_copy(ev_src(0), ev0, sev0)

    for s in range(n_sub):
        pltpu.make_async_copy(rowi.at[pl.ds(0, S_SUB)],
                              ris[s % 2], sris[s % 2]).wait()
        if s + 1 < n_sub:
            nb = pl.multiple_of(tile_base + (s + 1) * S_SUB, 8)
            pltpu.async_copy(rowi.at[pl.ds(nb, S_SUB)],
                             ris[(s + 1) % 2], sris[(s + 1) % 2])
        for c in range(DIM):
            k = s * DIM + c
            pltpu.make_async_copy(em0.at[pl.ds(0, S_SUB)],
                                  evs[k % 2], sevs[k % 2]).wait()
            if k + 1 < n_sub * DIM:
                pltpu.async_copy(ev_src(k + 1),
                                 evs[(k + 1) % 2], sevs[(k + 1) % 2])
            pltpu.sync_copy(evs[k % 2], accs[c].at[ris[s % 2]], add=True)

    plsc.subcore_barrier()
    for c in range(DIM):
        abase = pl.multiple_of(sid * SW, 8)
        obase = pl.multiple_of(cc * ACC_FLAT + c * N_PAD + sid * SW, 8)
        pltpu.sync_copy(accs[c].at[pl.ds(abase, SW)], zbuf)
        pltpu.sync_copy(zbuf, out.at[pl.ds(obase, SW)])


@functools.partial(
    pl.kernel,
    out_type=jax.ShapeDtypeStruct((NC * N_PAD,), jnp.float32),
    mesh=_mesh,
    compiler_params=pltpu.CompilerParams(needs_layout_passes=False),
    scratch_types=[
        pltpu.VMEM_SHARED((CACC_FLAT,), jnp.float32),
        pltpu.VMEM((CACC_CHUNK,), jnp.float32),
        pltpu.VMEM((S_SUB,), jnp.int32),
        pltpu.VMEM((S_SUB,), jnp.float32),
    ],
)
def _sc_count(rowi, out, acc, zbuf, ri, ones):
    cc = lax.axis_index("c")
    sid = lax.axis_index("s")

    def zfill(i, c2):
        zbuf[pl.ds(i * LANES, LANES)] = jnp.zeros((LANES,), jnp.float32)
        return c2

    lax.fori_loop(0, CACC_CHUNK // LANES, zfill, 0)
    pltpu.sync_copy(zbuf, acc.at[pl.ds(sid * CACC_CHUNK, CACC_CHUNK)])

    def ones_body(i, c2):
        ones[pl.ds(i * LANES, LANES)] = jnp.ones((LANES,), jnp.float32)
        return c2

    lax.fori_loop(0, S_SUB // LANES, ones_body, 0)
    plsc.subcore_barrier()

    def sub_body(s, carry):
        base = (cc * NS + sid) * S_PER + s * S_SUB
        pltpu.sync_copy(rowi.at[pl.ds(base, S_SUB)], ri)
        pltpu.sync_copy(ones, acc.at[ri], add=True)
        return carry

    lax.fori_loop(0, S_PER // S_SUB, sub_body, 0)
    plsc.subcore_barrier()

    abase = pl.multiple_of(sid * CACC_CHUNK, 8)
    cbase = pl.multiple_of(cc * N_PAD + sid * CACC_CHUNK, 8)
    pltpu.sync_copy(acc.at[pl.ds(abase, CACC_CHUNK)], zbuf)
    pltpu.sync_copy(zbuf, out.at[pl.ds(cbase, CACC_CHUNK)])


def _edge_mlp_body(xd0_ref, xd1_ref, xd2_ref,
                   ea0_ref, ea1_ref, ea2_ref, ea3_ref,
                   w1a_ref, w2_ref, o0_ref, o1_ref, o2_ref):
    xd0 = xd0_ref[...]
    xd1 = xd1_ref[...]
    xd2 = xd2_ref[...]
    r = jnp.sqrt(xd0 * xd0 + xd1 * xd1 + xd2 * xd2)
    e_in = jnp.stack([r, ea0_ref[...], ea1_ref[...], ea2_ref[...],
                      ea3_ref[...], jnp.ones_like(r)])
    h = lax.dot_general(w1a_ref[...], e_in, (((0,), (0,)), ((), ())),
                        preferred_element_type=jnp.float32)
    hh = 0.5 * h
    h = hh + hh * jnp.tanh(hh)
    eo = lax.dot_general(w2_ref[...], h, (((0,), (0,)), ((), ())),
                         preferred_element_type=jnp.float32)
    t = jnp.tanh(eo)[0]
    o0_ref[...] = xd0 * t
    o1_ref[...] = xd1 * t
    o2_ref[...] = xd2 * t


def _edge_mlp(xdf, eaf, w1a, w2):
    espec = jax.ShapeDtypeStruct((E_PAD,), jnp.float32)
    return pl.pallas_call(
        _edge_mlp_body,
        grid=(NEB,),
        in_specs=[
            pl.BlockSpec((EB,), lambda i: (i,)),
            pl.BlockSpec((EB,), lambda i: (NEB + i,)),
            pl.BlockSpec((EB,), lambda i: (2 * NEB + i,)),
            pl.BlockSpec((EB,), lambda i: (i,)),
            pl.BlockSpec((EB,), lambda i: (NEB + i,)),
            pl.BlockSpec((EB,), lambda i: (2 * NEB + i,)),
            pl.BlockSpec((EB,), lambda i: (3 * NEB + i,)),
            pl.BlockSpec((1 + EA + 1, NF), lambda i: (0, 0)),
            pl.BlockSpec((NF, 1), lambda i: (0, 0)),
        ],
        out_specs=(
            pl.BlockSpec((EB,), lambda i: (i,)),
            pl.BlockSpec((EB,), lambda i: (i,)),
            pl.BlockSpec((EB,), lambda i: (i,)),
        ),
        out_shape=(espec, espec, espec),
    )(xdf, xdf, xdf, eaf, eaf, eaf, eaf, w1a, w2)


def _inv_cnt_body(p0_ref, p1_ref, out_ref):
    s = p0_ref[...] + p1_ref[...]
    out_ref[...] = 1.0 / jnp.maximum(s, 1.0)


def _inv_cnt(cntf):
    return pl.pallas_call(
        _inv_cnt_body,
        grid=(NNB,),
        in_specs=[
            pl.BlockSpec((NB,), lambda i: (i,)),
            pl.BlockSpec((NB,), lambda i: (NNB + i,)),
        ],
        out_specs=pl.BlockSpec((NB,), lambda i: (i,)),
        out_shape=jax.ShapeDtypeStruct((N_PAD,), jnp.float32),
    )(cntf, cntf)


def _vscale_body(vn_ref, w1_ref, b1_ref, w2_ref, b2_ref, out_ref):
    vn = vn_ref[...][None, :]
    w1 = w1_ref[...]
    b1 = b1_ref[...]
    w2 = w2_ref[...]
    b2 = b2_ref[...]
    rows = []
    for l in range(NL):
        h = w1[l][:, None] * vn + b1[l][:, None]
        hh = 0.5 * h
        h = hh + hh * jnp.tanh(hh)
        rows.append(jnp.sum(h * w2[l][:, None], axis=0) + b2[l, 0])
    out_ref[...] = jnp.stack(rows)


def _vscale(vnf, w1s, b1s, w2s, b2s):
    return pl.pallas_call(
        _vscale_body,
        grid=(NNB,),
        in_specs=[
            pl.BlockSpec((NB,), lambda n: (n,)),
            pl.BlockSpec((NL, NF), lambda n: (0, 0)),
            pl.BlockSpec((NL, NF), lambda n: (0, 0)),
            pl.BlockSpec((NL, NF), lambda n: (0, 0)),
            pl.BlockSpec((NL, 1), lambda n: (0, 0)),
        ],
        out_specs=pl.BlockSpec((NL, NB), lambda n: (0, n)),
        out_shape=jax.ShapeDtypeStruct((NL, N_PAD), jnp.float32),
    )(vnf, w1s, b1s, w2s, b2s)


def _update_body(layer, x_ref, p0_ref, p1_ref, inv_ref, vel_ref, vs_ref,
                 out_ref):
    agg = (p0_ref[...] + p1_ref[...]) * inv_ref[...]
    vs = vs_ref[...][layer]
    out_ref[...] = x_ref[...] + agg + vel_ref[...] * vs


def _update(xTf, partf, invf, velTf, vsf, layer):
    nblk = DIM * NNB
    return pl.pallas_call(
        functools.partial(_update_body, layer),
        grid=(nblk,),
        in_specs=[
            pl.BlockSpec((NB,), lambda i: (i,)),
            pl.BlockSpec((NB,), lambda i: (i,)),
            pl.BlockSpec((NB,), lambda i: (nblk + i,)),
            pl.BlockSpec((NB,), lambda i: (i % NNB,)),
            pl.BlockSpec((NB,), lambda i: (i,)),
            pl.BlockSpec((NL, NB), lambda i: (0, i % NNB)),
        ],
        out_specs=pl.BlockSpec((NB,), lambda i: (i,)),
        out_shape=jax.ShapeDtypeStruct((DIM * N_PAD,), jnp.float32),
    )(xTf, partf, partf, invf, velTf, vsf)


def kernel(vel_norm, x, edges, vel, edge_attr, params):
    row = edges[0]
    col = edges[1]

    pad = N_PAD - N_NODES
    xTf = jnp.pad(x.T, ((0, 0), (0, pad))).reshape(-1)
    velTf = jnp.pad(vel.T, ((0, 0), (0, pad))).reshape(-1)
    vnf = jnp.pad(vel_norm[:, 0], (0, pad))
    eaf = jnp.pad(edge_attr.T, ((0, 0), (0, E_PAD - N_EDGES))).reshape(-1)

    cntf = _sc_count(row)
    invf = _inv_cnt(cntf)

    w1s = jnp.stack([p['vW1'][0] for p in params])
    b1s = jnp.stack([p['vb1'] for p in params])
    w2s = jnp.stack([p['vW2'][:, 0] for p in params])
    b2s = jnp.stack([p['vb2'] for p in params])
    vsf = _vscale(vnf, w1s, b1s, w2s, b2s)

    for l, p in enumerate(params):
        w1a = jnp.concatenate([p['pW1'], p['pb1'][None, :]], axis=0)
        xdf = _sc_gather_diff(xTf, row, col)
        em0, em1, em2 = _edge_mlp(xdf, eaf, w1a, p['pW2'])
        partf = _sc_scatter(em0, em1, em2, row)
        xTf = _update(xTf, partf, invf, velTf, vsf, l)

    return xTf.reshape(DIM, N_PAD)[:, :N_NODES].T
